# baseline (speedup 1.0000x reference)
import functools

import jax
import jax.numpy as jnp
from jax import lax
from jax.experimental import pallas as pl
from jax.experimental.pallas import tpu as pltpu
from jax.experimental.pallas import tpu_sc as plsc

N = 10000
NP = 10240
E = 320000

NC = 2
NS = 16
L = 16

EDGES_PER_TILE = E // (NC * NS)
EPT_PAD = 10240
CHUNK = 64
NCHUNK = EPT_PAD // CHUNK
NBUF = 4
ROWS_PER_TILE = NP // NS
ZROWS = 16


def _splat(ew_v, lane):
  idx = jnp.full((L,), lane, jnp.int32)
  return lax.gather(
      ew_v, idx[:, None],
      lax.GatherDimensionNumbers(offset_dims=(), collapsed_slice_dims=(0,),
                                 start_index_map=(0,)),
      (1,), mode=lax.GatherScatterMode.PROMISE_IN_BOUNDS)


def _spmm_body(F, edg_hbm, dst_hbm, sup_hbm, out_hbm,
               edg0, edg1, edg2, edg3, dst0, dst1, dst2, dst3,
               rows0, rows1, rows2, rows3, zbuf, accum,
               gsem0, gsem1, gsem2, gsem3, ssem0, ssem1, ssem2, ssem3):
  c = lax.axis_index("c")
  s = lax.axis_index("s")
  edg = [edg0, edg1, edg2, edg3]
  dstb = [dst0, dst1, dst2, dst3]
  rows = [rows0, rows1, rows2, rows3]
  gsem = [gsem0, gsem1, gsem2, gsem3]
  ssem = [ssem0, ssem1, ssem2, ssem3]

  for r in range(ZROWS):
    for q in range(F // L):
      zbuf[r, pl.ds(q * L, L)] = jnp.zeros((L,), jnp.float32)

  def zero_rows(k, _):
    pltpu.sync_copy(zbuf, accum.at[pl.ds(s * ROWS_PER_TILE + k * ZROWS,
                                         ZROWS)])
    return 0
  lax.fori_loop(0, ROWS_PER_TILE // ZROWS, zero_rows, 0)
  plsc.subcore_barrier()

  def scale(b):
    def scale_group(g, _):
      ew_v = plsc.bitcast(edg[b][1, pl.ds(g * L, L)], jnp.float32)
      for l in range(L):
        spl = _splat(ew_v, l)
        e = g * L + l
        for q in range(F // L):
          rows[b][e, pl.ds(q * L, L)] = rows[b][e, pl.ds(q * L, L)] * spl
      return 0
    lax.fori_loop(0, CHUNK // L, scale_group, 0)

  def stage_and_gather(j, b):
    pltpu.sync_copy(edg_hbm.at[c, s, j], edg[b])
    pltpu.sync_copy(dst_hbm.at[c, s, j], dstb[b])
    pltpu.make_async_copy(sup_hbm.at[edg[b].at[0]], rows[b], gsem[b]).start()

  def gather_wait(b):
    pltpu.make_async_copy(sup_hbm.at[edg[b].at[0]], rows[b], gsem[b]).wait()

  def scatter_start(b):
    pltpu.async_copy(rows[b], accum.at[dstb[b]], ssem[b], add=True)

  def scatter_wait(b):
    pltpu.make_async_copy(rows[b], accum.at[dstb[b]], ssem[b]).wait()

  stage_and_gather(0, 0)
  stage_and_gather(1, 1)

  def ring_body(k, _):
    for b in range(NBUF):
      j = k * NBUF + b
      jn = j + 2
      slot = (b + 2) % NBUF

      @pl.when(jn < NCHUNK)
      def _():
        @pl.when(jn >= NBUF)
        def _():
          scatter_wait(slot)
        stage_and_gather(jn, slot)

      gather_wait(b)
      scale(b)
      scatter_start(b)
    return 0

  lax.fori_loop(0, NCHUNK // NBUF, ring_body, 0)
  for b in range(NBUF):
    scatter_wait(b)
  plsc.subcore_barrier()

  pltpu.sync_copy(accum.at[pl.ds(s * ROWS_PER_TILE, ROWS_PER_TILE)],
                  out_hbm.at[c, pl.ds(s * ROWS_PER_TILE, ROWS_PER_TILE)])


def _make_spmm(F):
  mesh = plsc.VectorSubcoreMesh(core_axis_name="c", subcore_axis_name="s")
  return pl.kernel(
      functools.partial(_spmm_body, F),
      out_type=jax.ShapeDtypeStruct((NC, NP, F), jnp.float32),
      mesh=mesh,
      scratch_types=(
          [pltpu.VMEM((2, CHUNK), jnp.int32)] * NBUF +
          [pltpu.VMEM((CHUNK,), jnp.int32)] * NBUF +
          [pltpu.VMEM((CHUNK, F), jnp.float32)] * NBUF +
          [pltpu.VMEM((ZROWS, F), jnp.float32),
           pltpu.VMEM_SHARED((NP, F), jnp.float32)] +
          [pltpu.SemaphoreType.DMA] * (2 * NBUF)
      ),
      compiler_params=pltpu.CompilerParams(use_tc_tiling_on_sc=False,
                                           needs_layout_passes=False),
      name=f"spmm_sc_f{F}",
  )


_spmm_128 = _make_spmm(128)
_spmm_64 = _make_spmm(64)
_spmm_48 = _make_spmm(48)


def _spmm(sup, edges, dsts, F):
  if F == 128:
    return _spmm_128(edges, dsts, sup)
  if F == 64:
    return _spmm_64(edges, dsts, sup)
  return _spmm_48(edges, dsts, sup)


BLK = 1024


def _mm_body(x_ref, w_ref, o_ref):
  o_ref[...] = jnp.dot(x_ref[...], w_ref[...],
                       preferred_element_type=jnp.float32)


def _mm(x, w):
  n, k = x.shape
  m = w.shape[1]
  return pl.pallas_call(
      _mm_body,
      grid=(n // BLK,),
      in_specs=[pl.BlockSpec((BLK, k), lambda i: (i, 0)),
                pl.BlockSpec((k, m), lambda i: (0, 0))],
      out_specs=pl.BlockSpec((BLK, m), lambda i: (i, 0)),
      out_shape=jax.ShapeDtypeStruct((n, m), jnp.float32),
  )(x, w)


def _fuse1_body(p_ref, b1_ref, w2_ref, ew_ref, eb_ref, s2_ref, o2_ref):
  h = jax.nn.relu(p_ref[0] + p_ref[1] + b1_ref[...])
  s2_ref[...] = jnp.dot(h, w2_ref[...], preferred_element_type=jnp.float32)
  o2_ref[...] = jnp.dot(h, ew_ref[...],
                        preferred_element_type=jnp.float32) + eb_ref[...]


def _fuse1(p, b1, w2, encw, encb):
  return pl.pallas_call(
      _fuse1_body,
      grid=(NP // BLK,),
      in_specs=[pl.BlockSpec((2, BLK, 128), lambda i: (0, i, 0)),
                pl.BlockSpec((1, 128), lambda i: (0, 0)),
                pl.BlockSpec((128, 64), lambda i: (0, 0)),
                pl.BlockSpec((128, 16), lambda i: (0, 0)),
                pl.BlockSpec((1, 16), lambda i: (0, 0))],
      out_specs=[pl.BlockSpec((BLK, 64), lambda i: (i, 0)),
                 pl.BlockSpec((BLK, 16), lambda i: (i, 0))],
      out_shape=[jax.ShapeDtypeStruct((NP, 64), jnp.float32),
                 jax.ShapeDtypeStruct((NP, 16), jnp.float32)],
  )(p, b1, w2, encw, encb)


def _fuse2_body(p_ref, b2_ref, w3_ref, s3_ref):
  h = jax.nn.relu(p_ref[0] + p_ref[1] + b2_ref[...])
  s3_ref[...] = jnp.dot(h, w3_ref[...], preferred_element_type=jnp.float32)


def _fuse2(p, b2, w3p):
  return pl.pallas_call(
      _fuse2_body,
      grid=(NP // BLK,),
      in_specs=[pl.BlockSpec((2, BLK, 64), lambda i: (0, i, 0)),
                pl.BlockSpec((1, 64), lambda i: (0, 0)),
                pl.BlockSpec((64, 48), lambda i: (0, 0))],
      out_specs=pl.BlockSpec((BLK, 48), lambda i: (i, 0)),
      out_shape=jax.ShapeDtypeStruct((NP, 48), jnp.float32),
  )(p, b2, w3p)


def _final_body(p_ref, b3_ref, o_ref):
  z = p_ref[0] + p_ref[1] + b3_ref[...]
  m = jnp.max(z, axis=1, keepdims=True)
  lse = jnp.log(jnp.sum(jnp.exp(z - m), axis=1, keepdims=True))
  o_ref[...] = z - m - lse


def _final(p, b3p):
  return pl.pallas_call(
      _final_body,
      grid=(NP // BLK,),
      in_specs=[pl.BlockSpec((2, BLK, 48), lambda i: (0, i, 0)),
                pl.BlockSpec((1, 48), lambda i: (0, 0))],
      out_specs=pl.BlockSpec((BLK, 48), lambda i: (i, 0)),
      out_shape=jax.ShapeDtypeStruct((NP, 48), jnp.float32),
  )(p, b3p)


@jax.jit
def kernel(x, edge_index, edge_weight, W1, b1, W2, b2, W3, b3, encW, encb):
  pad = EPT_PAD - EDGES_PER_TILE
  src2 = jnp.pad(edge_index[0].reshape(NC * NS, EDGES_PER_TILE),
                 ((0, 0), (0, pad)))
  dst2 = jnp.pad(edge_index[1].reshape(NC * NS, EDGES_PER_TILE),
                 ((0, 0), (0, pad)), constant_values=NP - 1)
  ew2 = jnp.pad(lax.bitcast_convert_type(edge_weight, jnp.int32).reshape(
      NC * NS, EDGES_PER_TILE), ((0, 0), (0, pad)))
  edges = jnp.concatenate(
      [src2.reshape(NC, NS, NCHUNK, 1, CHUNK),
       ew2.reshape(NC, NS, NCHUNK, 1, CHUNK)], axis=3)
  dsts = dst2.reshape(NC, NS, NCHUNK, CHUNK)

  xp = jnp.pad(x, ((0, NP - N), (0, 0)))
  w3p = jnp.pad(W3, ((0, 0), (0, 8)))
  b3p = jnp.concatenate([b3, jnp.full((8,), -1e30, jnp.float32)])

  sup1 = _mm(xp, W1)
  p1 = _spmm(sup1, edges, dsts, 128)
  sup2, out2 = _fuse1(p1, b1.reshape(1, -1), W2, encW, encb.reshape(1, -1))
  p2 = _spmm(sup2, edges, dsts, 64)
  sup3 = _fuse2(p2, b2.reshape(1, -1), w3p)
  p3 = _spmm(sup3, edges, dsts, 48)
  out1 = _final(p3, b3p.reshape(1, -1))
  return (out1[:N, :40], out2[:N])


# speedup vs baseline: 1.1147x; 1.1147x over previous
import functools

import jax
import jax.numpy as jnp
from jax import lax
from jax.experimental import pallas as pl
from jax.experimental.pallas import tpu as pltpu
from jax.experimental.pallas import tpu_sc as plsc

N = 10000
NP = 10240
E = 320000

NC = 2
NS = 16
L = 16

EDGES_PER_TILE = E // (NC * NS)
EPT_PAD = 10240
CHUNK = 80
NCHUNK = EPT_PAD // CHUNK
NBUF = 2
ROWS_PER_TILE = NP // NS
ZROWS = 16


def _splat(ew_v, lane):
  idx = jnp.full((L,), lane, jnp.int32)
  return lax.gather(
      ew_v, idx[:, None],
      lax.GatherDimensionNumbers(offset_dims=(), collapsed_slice_dims=(0,),
                                 start_index_map=(0,)),
      (1,), mode=lax.GatherScatterMode.PROMISE_IN_BOUNDS)


def _spmm_body(F, edg_hbm, dst_hbm, sup_hbm, out_hbm,
               srcb, ewb, dst0, dst1, rows0, rows1, zbuf, accum,
               gsem0, gsem1):
  c = lax.axis_index("c")
  s = lax.axis_index("s")
  base = (c * NS + s) * EPT_PAD
  dstb = [dst0, dst1]
  rows = [rows0, rows1]
  gsem = [gsem0, gsem1]

  pltpu.sync_copy(edg_hbm.at[0, pl.ds(base, EPT_PAD)], srcb)
  pltpu.sync_copy(edg_hbm.at[1, pl.ds(base, EPT_PAD)], ewb)

  for r in range(ZROWS):
    for q in range(F // L):
      zbuf[r, pl.ds(q * L, L)] = jnp.zeros((L,), jnp.float32)

  def zero_rows(k, _):
    pltpu.sync_copy(zbuf, accum.at[pl.ds(s * ROWS_PER_TILE + k * ZROWS,
                                         ZROWS)])
    return 0
  lax.fori_loop(0, ROWS_PER_TILE // ZROWS, zero_rows, 0)
  plsc.subcore_barrier()

  def scale(b, j):
    def scale_group(g, _):
      ew_v = plsc.bitcast(ewb[pl.ds(j * CHUNK + g * L, L)], jnp.float32)
      for l in range(L):
        spl = _splat(ew_v, l)
        e = g * L + l
        for q in range(F // L):
          rows[b][e, pl.ds(q * L, L)] = rows[b][e, pl.ds(q * L, L)] * spl
      return 0
    lax.fori_loop(0, CHUNK // L, scale_group, 0)

  def gather_start(j, b):
    pltpu.sync_copy(dst_hbm.at[c, s, j], dstb[b])
    pltpu.make_async_copy(
        sup_hbm.at[srcb.at[pl.ds(j * CHUNK, CHUNK)]], rows[b], gsem[b]
    ).start()

  def gather_wait(j, b):
    pltpu.make_async_copy(
        sup_hbm.at[srcb.at[pl.ds(j * CHUNK, CHUNK)]], rows[b], gsem[b]
    ).wait()

  gather_start(0, 0)

  def pair_body(k, _):
    j0 = 2 * k
    gather_start(j0 + 1, 1)
    gather_wait(j0, 0)
    scale(0, j0)
    pltpu.sync_copy(rows[0], accum.at[dstb[0]], add=True)

    @pl.when(j0 + 2 < NCHUNK)
    def _():
      gather_start(j0 + 2, 0)

    gather_wait(j0 + 1, 1)
    scale(1, j0 + 1)
    pltpu.sync_copy(rows[1], accum.at[dstb[1]], add=True)
    return 0

  lax.fori_loop(0, NCHUNK // 2, pair_body, 0)
  plsc.subcore_barrier()

  pltpu.sync_copy(accum.at[pl.ds(s * ROWS_PER_TILE, ROWS_PER_TILE)],
                  out_hbm.at[c, pl.ds(s * ROWS_PER_TILE, ROWS_PER_TILE)])


def _make_spmm(F):
  mesh = plsc.VectorSubcoreMesh(core_axis_name="c", subcore_axis_name="s")
  return pl.kernel(
      functools.partial(_spmm_body, F),
      out_type=jax.ShapeDtypeStruct((NC, NP, F), jnp.float32),
      mesh=mesh,
      scratch_types=(
          [pltpu.VMEM((EPT_PAD,), jnp.int32),
           pltpu.VMEM((EPT_PAD,), jnp.int32)] +
          [pltpu.VMEM((CHUNK,), jnp.int32)] * NBUF +
          [pltpu.VMEM((CHUNK, F), jnp.float32)] * NBUF +
          [pltpu.VMEM((ZROWS, F), jnp.float32),
           pltpu.VMEM_SHARED((NP, F), jnp.float32)] +
          [pltpu.SemaphoreType.DMA] * NBUF
      ),
      compiler_params=pltpu.CompilerParams(use_tc_tiling_on_sc=False,
                                           needs_layout_passes=False),
      name=f"spmm_sc_f{F}",
  )


_spmm_128 = _make_spmm(128)
_spmm_64 = _make_spmm(64)
_spmm_48 = _make_spmm(48)


def _spmm(sup, edges, dsts, F):
  if F == 128:
    return _spmm_128(edges, dsts, sup)
  if F == 64:
    return _spmm_64(edges, dsts, sup)
  return _spmm_48(edges, dsts, sup)


BLK = 1024


def _mm_body(x_ref, w_ref, o_ref):
  o_ref[...] = jnp.dot(x_ref[...], w_ref[...],
                       preferred_element_type=jnp.float32)


def _mm(x, w):
  n, k = x.shape
  m = w.shape[1]
  return pl.pallas_call(
      _mm_body,
      grid=(n // BLK,),
      in_specs=[pl.BlockSpec((BLK, k), lambda i: (i, 0)),
                pl.BlockSpec((k, m), lambda i: (0, 0))],
      out_specs=pl.BlockSpec((BLK, m), lambda i: (i, 0)),
      out_shape=jax.ShapeDtypeStruct((n, m), jnp.float32),
  )(x, w)


def _fuse1_body(p_ref, b1_ref, w2_ref, ew_ref, eb_ref, s2_ref, o2_ref):
  h = jax.nn.relu(p_ref[0] + p_ref[1] + b1_ref[...])
  s2_ref[...] = jnp.dot(h, w2_ref[...], preferred_element_type=jnp.float32)
  o2_ref[...] = jnp.dot(h, ew_ref[...],
                        preferred_element_type=jnp.float32) + eb_ref[...]


def _fuse1(p, b1, w2, encw, encb):
  return pl.pallas_call(
      _fuse1_body,
      grid=(NP // BLK,),
      in_specs=[pl.BlockSpec((2, BLK, 128), lambda i: (0, i, 0)),
                pl.BlockSpec((1, 128), lambda i: (0, 0)),
                pl.BlockSpec((128, 64), lambda i: (0, 0)),
                pl.BlockSpec((128, 16), lambda i: (0, 0)),
                pl.BlockSpec((1, 16), lambda i: (0, 0))],
      out_specs=[pl.BlockSpec((BLK, 64), lambda i: (i, 0)),
                 pl.BlockSpec((BLK, 16), lambda i: (i, 0))],
      out_shape=[jax.ShapeDtypeStruct((NP, 64), jnp.float32),
                 jax.ShapeDtypeStruct((NP, 16), jnp.float32)],
  )(p, b1, w2, encw, encb)


def _fuse2_body(p_ref, b2_ref, w3_ref, s3_ref):
  h = jax.nn.relu(p_ref[0] + p_ref[1] + b2_ref[...])
  s3_ref[...] = jnp.dot(h, w3_ref[...], preferred_element_type=jnp.float32)


def _fuse2(p, b2, w3p):
  return pl.pallas_call(
      _fuse2_body,
      grid=(NP // BLK,),
      in_specs=[pl.BlockSpec((2, BLK, 64), lambda i: (0, i, 0)),
                pl.BlockSpec((1, 64), lambda i: (0, 0)),
                pl.BlockSpec((64, 48), lambda i: (0, 0))],
      out_specs=pl.BlockSpec((BLK, 48), lambda i: (i, 0)),
      out_shape=jax.ShapeDtypeStruct((NP, 48), jnp.float32),
  )(p, b2, w3p)


def _final_body(p_ref, b3_ref, o_ref):
  z = p_ref[0] + p_ref[1] + b3_ref[...]
  m = jnp.max(z, axis=1, keepdims=True)
  lse = jnp.log(jnp.sum(jnp.exp(z - m), axis=1, keepdims=True))
  o_ref[...] = z - m - lse


def _final(p, b3p):
  return pl.pallas_call(
      _final_body,
      grid=(NP // BLK,),
      in_specs=[pl.BlockSpec((2, BLK, 48), lambda i: (0, i, 0)),
                pl.BlockSpec((1, 48), lambda i: (0, 0))],
      out_specs=pl.BlockSpec((BLK, 48), lambda i: (i, 0)),
      out_shape=jax.ShapeDtypeStruct((NP, 48), jnp.float32),
  )(p, b3p)


@jax.jit
def kernel(x, edge_index, edge_weight, W1, b1, W2, b2, W3, b3, encW, encb):
  pad = EPT_PAD - EDGES_PER_TILE
  src2 = jnp.pad(edge_index[0].reshape(NC * NS, EDGES_PER_TILE),
                 ((0, 0), (0, pad)))
  dst2 = jnp.pad(edge_index[1].reshape(NC * NS, EDGES_PER_TILE),
                 ((0, 0), (0, pad)), constant_values=NP - 1)
  ew2 = jnp.pad(lax.bitcast_convert_type(edge_weight, jnp.int32).reshape(
      NC * NS, EDGES_PER_TILE), ((0, 0), (0, pad)))
  edges = jnp.stack([src2.reshape(-1), ew2.reshape(-1)])
  dsts = dst2.reshape(NC, NS, NCHUNK, CHUNK)

  xp = jnp.pad(x, ((0, NP - N), (0, 0)))
  w3p = jnp.pad(W3, ((0, 0), (0, 8)))
  b3p = jnp.concatenate([b3, jnp.full((8,), -1e30, jnp.float32)])

  sup1 = _mm(xp, W1)
  p1 = _spmm(sup1, edges, dsts, 128)
  sup2, out2 = _fuse1(p1, b1.reshape(1, -1), W2, encW, encb.reshape(1, -1))
  p2 = _spmm(sup2, edges, dsts, 64)
  sup3 = _fuse2(p2, b2.reshape(1, -1), w3p)
  p3 = _spmm(sup3, edges, dsts, 48)
  out1 = _final(p3, b3p.reshape(1, -1))
  return (out1[:N, :40], out2[:N])


# speedup vs baseline: 1.1615x; 1.0420x over previous
import functools

import jax
import jax.numpy as jnp
from jax import lax
from jax.experimental import pallas as pl
from jax.experimental.pallas import tpu as pltpu
from jax.experimental.pallas import tpu_sc as plsc

N = 10000
NP = 10240
E = 320000

NC = 2
NS = 16
L = 16

EDGES_PER_TILE = E // (NC * NS)
EPT_PAD = 10240
NBUF = 2
ROWS_PER_TILE = NP // NS
ZROWS = 16


def _chunk_for(F):
  return 80 if F == 128 else 128


def _splat(ew_v, lane):
  idx = jnp.full((L,), lane, jnp.int32)
  return lax.gather(
      ew_v, idx[:, None],
      lax.GatherDimensionNumbers(offset_dims=(), collapsed_slice_dims=(0,),
                                 start_index_map=(0,)),
      (1,), mode=lax.GatherScatterMode.PROMISE_IN_BOUNDS)


def _spmm_body(F, CHUNK, edg_hbm, dst_hbm, sup_hbm, out_hbm,
               srcb, ewb, dst0, dst1, rows0, rows1, zbuf, accum,
               gsem0, gsem1, zsem):
  NCHUNK = EPT_PAD // CHUNK
  c = lax.axis_index("c")
  s = lax.axis_index("s")
  base = (c * NS + s) * EPT_PAD
  dstb = [dst0, dst1]
  rows = [rows0, rows1]
  gsem = [gsem0, gsem1]

  for r in range(ZROWS):
    for q in range(F // L):
      zbuf[r, pl.ds(q * L, L)] = jnp.zeros((L,), jnp.float32)

  def zero_start(k, _):
    pltpu.make_async_copy(
        zbuf, accum.at[pl.ds(s * ROWS_PER_TILE + k * ZROWS, ZROWS)],
        zsem).start()
    return 0
  lax.fori_loop(0, ROWS_PER_TILE // ZROWS, zero_start, 0)

  pltpu.sync_copy(edg_hbm.at[0, pl.ds(base, EPT_PAD)], srcb)
  pltpu.sync_copy(edg_hbm.at[1, pl.ds(base, EPT_PAD)], ewb)

  def zero_wait(k, _):
    pltpu.make_async_copy(
        zbuf, accum.at[pl.ds(s * ROWS_PER_TILE + k * ZROWS, ZROWS)],
        zsem).wait()
    return 0
  lax.fori_loop(0, ROWS_PER_TILE // ZROWS, zero_wait, 0)
  plsc.subcore_barrier()

  def scale(b, j):
    def scale_group(g, _):
      ew_v = plsc.bitcast(ewb[pl.ds(j * CHUNK + g * L, L)], jnp.float32)
      for l in range(L):
        spl = _splat(ew_v, l)
        e = g * L + l
        for q in range(F // L):
          rows[b][e, pl.ds(q * L, L)] = rows[b][e, pl.ds(q * L, L)] * spl
      return 0
    lax.fori_loop(0, CHUNK // L, scale_group, 0)

  def gather_start(j, b):
    pltpu.sync_copy(dst_hbm.at[c, s, j], dstb[b])
    pltpu.make_async_copy(
        sup_hbm.at[srcb.at[pl.ds(j * CHUNK, CHUNK)]], rows[b], gsem[b]
    ).start()

  def gather_wait(j, b):
    pltpu.make_async_copy(
        sup_hbm.at[srcb.at[pl.ds(j * CHUNK, CHUNK)]], rows[b], gsem[b]
    ).wait()

  gather_start(0, 0)

  def pair_body(k, _):
    j0 = 2 * k
    gather_start(j0 + 1, 1)
    gather_wait(j0, 0)
    scale(0, j0)
    pltpu.sync_copy(rows[0], accum.at[dstb[0]], add=True)

    @pl.when(j0 + 2 < NCHUNK)
    def _():
      gather_start(j0 + 2, 0)

    gather_wait(j0 + 1, 1)
    scale(1, j0 + 1)
    pltpu.sync_copy(rows[1], accum.at[dstb[1]], add=True)
    return 0

  lax.fori_loop(0, NCHUNK // 2, pair_body, 0)
  plsc.subcore_barrier()

  pltpu.sync_copy(accum.at[pl.ds(s * ROWS_PER_TILE, ROWS_PER_TILE)],
                  out_hbm.at[c, pl.ds(s * ROWS_PER_TILE, ROWS_PER_TILE)])


def _make_spmm(F):
  CHUNK = _chunk_for(F)
  mesh = plsc.VectorSubcoreMesh(core_axis_name="c", subcore_axis_name="s")
  return pl.kernel(
      functools.partial(_spmm_body, F, CHUNK),
      out_type=jax.ShapeDtypeStruct((NC, NP, F), jnp.float32),
      mesh=mesh,
      scratch_types=(
          [pltpu.VMEM((EPT_PAD,), jnp.int32),
           pltpu.VMEM((EPT_PAD,), jnp.int32)] +
          [pltpu.VMEM((CHUNK,), jnp.int32)] * NBUF +
          [pltpu.VMEM((CHUNK, F), jnp.float32)] * NBUF +
          [pltpu.VMEM((ZROWS, F), jnp.float32),
           pltpu.VMEM_SHARED((NP, F), jnp.float32)] +
          [pltpu.SemaphoreType.DMA] * (NBUF + 1)
      ),
      compiler_params=pltpu.CompilerParams(use_tc_tiling_on_sc=False,
                                           needs_layout_passes=False),
      name=f"spmm_sc_f{F}",
  )


_spmm_128 = _make_spmm(128)
_spmm_64 = _make_spmm(64)
_spmm_48 = _make_spmm(48)


def _spmm(sup, edges, dsts, F):
  if F == 128:
    return _spmm_128(edges, dsts, sup)
  if F == 64:
    return _spmm_64(edges, dsts, sup)
  return _spmm_48(edges, dsts, sup)


BLK = 1024


def _mm_body(x_ref, w_ref, o_ref):
  o_ref[...] = jnp.dot(x_ref[...], w_ref[...],
                       preferred_element_type=jnp.float32)


def _mm(x, w):
  n, k = x.shape
  m = w.shape[1]
  return pl.pallas_call(
      _mm_body,
      grid=(n // BLK,),
      in_specs=[pl.BlockSpec((BLK, k), lambda i: (i, 0)),
                pl.BlockSpec((k, m), lambda i: (0, 0))],
      out_specs=pl.BlockSpec((BLK, m), lambda i: (i, 0)),
      out_shape=jax.ShapeDtypeStruct((n, m), jnp.float32),
  )(x, w)


def _fuse1_body(p_ref, b1_ref, w2_ref, ew_ref, eb_ref, s2_ref, o2_ref):
  h = jax.nn.relu(p_ref[0] + p_ref[1] + b1_ref[...])
  s2_ref[...] = jnp.dot(h, w2_ref[...], preferred_element_type=jnp.float32)
  o2_ref[...] = jnp.dot(h, ew_ref[...],
                        preferred_element_type=jnp.float32) + eb_ref[...]


def _fuse1(p, b1, w2, encw, encb):
  return pl.pallas_call(
      _fuse1_body,
      grid=(NP // BLK,),
      in_specs=[pl.BlockSpec((2, BLK, 128), lambda i: (0, i, 0)),
                pl.BlockSpec((1, 128), lambda i: (0, 0)),
                pl.BlockSpec((128, 64), lambda i: (0, 0)),
                pl.BlockSpec((128, 16), lambda i: (0, 0)),
                pl.BlockSpec((1, 16), lambda i: (0, 0))],
      out_specs=[pl.BlockSpec((BLK, 64), lambda i: (i, 0)),
                 pl.BlockSpec((BLK, 16), lambda i: (i, 0))],
      out_shape=[jax.ShapeDtypeStruct((NP, 64), jnp.float32),
                 jax.ShapeDtypeStruct((NP, 16), jnp.float32)],
  )(p, b1, w2, encw, encb)


def _fuse2_body(p_ref, b2_ref, w3_ref, s3_ref):
  h = jax.nn.relu(p_ref[0] + p_ref[1] + b2_ref[...])
  s3_ref[...] = jnp.dot(h, w3_ref[...], preferred_element_type=jnp.float32)


def _fuse2(p, b2, w3p):
  return pl.pallas_call(
      _fuse2_body,
      grid=(NP // BLK,),
      in_specs=[pl.BlockSpec((2, BLK, 64), lambda i: (0, i, 0)),
                pl.BlockSpec((1, 64), lambda i: (0, 0)),
                pl.BlockSpec((64, 48), lambda i: (0, 0))],
      out_specs=pl.BlockSpec((BLK, 48), lambda i: (i, 0)),
      out_shape=jax.ShapeDtypeStruct((NP, 48), jnp.float32),
  )(p, b2, w3p)


def _final_body(p_ref, b3_ref, o_ref):
  z = p_ref[0] + p_ref[1] + b3_ref[...]
  m = jnp.max(z, axis=1, keepdims=True)
  lse = jnp.log(jnp.sum(jnp.exp(z - m), axis=1, keepdims=True))
  o_ref[...] = z - m - lse


def _final(p, b3p):
  return pl.pallas_call(
      _final_body,
      grid=(NP // BLK,),
      in_specs=[pl.BlockSpec((2, BLK, 48), lambda i: (0, i, 0)),
                pl.BlockSpec((1, 48), lambda i: (0, 0))],
      out_specs=pl.BlockSpec((BLK, 48), lambda i: (i, 0)),
      out_shape=jax.ShapeDtypeStruct((NP, 48), jnp.float32),
  )(p, b3p)


@jax.jit
def kernel(x, edge_index, edge_weight, W1, b1, W2, b2, W3, b3, encW, encb):
  pad = EPT_PAD - EDGES_PER_TILE
  src2 = jnp.pad(edge_index[0].reshape(NC * NS, EDGES_PER_TILE),
                 ((0, 0), (0, pad)))
  dst2 = jnp.pad(edge_index[1].reshape(NC * NS, EDGES_PER_TILE),
                 ((0, 0), (0, pad)), constant_values=NP - 1)
  ew2 = jnp.pad(lax.bitcast_convert_type(edge_weight, jnp.int32).reshape(
      NC * NS, EDGES_PER_TILE), ((0, 0), (0, pad)))
  edges = jnp.stack([src2.reshape(-1), ew2.reshape(-1)])
  c128 = _chunk_for(128)
  c64 = _chunk_for(64)
  dsts128 = dst2.reshape(NC, NS, EPT_PAD // c128, c128)
  dsts64 = dst2.reshape(NC, NS, EPT_PAD // c64, c64)

  xp = jnp.pad(x, ((0, NP - N), (0, 0)))
  w3p = jnp.pad(W3, ((0, 0), (0, 8)))
  b3p = jnp.concatenate([b3, jnp.full((8,), -1e30, jnp.float32)])

  sup1 = _mm(xp, W1)
  p1 = _spmm(sup1, edges, dsts128, 128)
  sup2, out2 = _fuse1(p1, b1.reshape(1, -1), W2, encW, encb.reshape(1, -1))
  p2 = _spmm(sup2, edges, dsts64, 64)
  sup3 = _fuse2(p2, b2.reshape(1, -1), w3p)
  p3 = _spmm(sup3, edges, dsts64, 48)
  out1 = _final(p3, b3p.reshape(1, -1))
  return (out1[:N, :40], out2[:N])
